# baseline (speedup 1.0000x reference)
import functools

import jax
import jax.numpy as jnp
from jax import lax
from jax.experimental import pallas as pl
from jax.experimental.pallas import tpu as pltpu
from jax.experimental.pallas import tpu_sc as plsc

_B = 4
_CHUNK_ROWS = 64


def kernel(batch_size, table):
    n, f = table.shape
    info = plsc.get_sparse_core_info()
    nw = info.num_cores * info.num_subcores
    rows_per_w = n // nw
    n_chunks = rows_per_w // _CHUNK_ROWS

    mesh = plsc.VectorSubcoreMesh(core_axis_name="c", subcore_axis_name="s")

    @functools.partial(
        pl.kernel,
        mesh=mesh,
        out_type=jax.ShapeDtypeStruct((_B, n, f), jnp.float32),
        scratch_types=[
            pltpu.VMEM((_CHUNK_ROWS, f), jnp.float32),
            pltpu.SemaphoreType.DMA,
        ],
    )
    def k(table_hbm, out_hbm, buf, out_sem):
        wid = lax.axis_index("s") * info.num_cores + lax.axis_index("c")
        base = wid * rows_per_w
        for c in range(n_chunks):
            r0 = base + c * _CHUNK_ROWS
            pltpu.sync_copy(table_hbm.at[pl.ds(r0, _CHUNK_ROWS)], buf)
            handles = [
                pltpu.async_copy(
                    buf, out_hbm.at[b, pl.ds(r0, _CHUNK_ROWS)], out_sem
                )
                for b in range(_B)
            ]
            for h in handles:
                h.wait()

    return k(table)


# speedup vs baseline: 1.0008x; 1.0008x over previous
import functools

import jax
import jax.numpy as jnp
from jax import lax
from jax.experimental import pallas as pl
from jax.experimental.pallas import tpu as pltpu
from jax.experimental.pallas import tpu_sc as plsc

_B = 4
_CHUNK_ROWS = 64


def kernel(batch_size, table):
    n, f = table.shape
    info = plsc.get_sparse_core_info()
    nw = info.num_cores * info.num_subcores
    rows_per_w = n // nw
    n_chunks = rows_per_w // _CHUNK_ROWS

    mesh = plsc.VectorSubcoreMesh(core_axis_name="c", subcore_axis_name="s")

    @functools.partial(
        pl.kernel,
        mesh=mesh,
        out_type=jax.ShapeDtypeStruct((_B, n, f), jnp.float32),
        scratch_types=[pltpu.VMEM((_CHUNK_ROWS, f), jnp.float32)],
    )
    def k(table_hbm, out_hbm, buf):
        wid = lax.axis_index("s") * info.num_cores + lax.axis_index("c")
        base = wid * rows_per_w
        for c in range(n_chunks):
            r0 = base + c * _CHUNK_ROWS
            pltpu.sync_copy(table_hbm.at[pl.ds(r0, _CHUNK_ROWS)], buf)
            for b in range(_B):
                pltpu.sync_copy(buf, out_hbm.at[b, pl.ds(r0, _CHUNK_ROWS)])

    return k(table)
